# baseline (speedup 1.0000x reference)
import jax
import jax.numpy as jnp
from jax import lax
from jax.experimental import pallas as pl
from jax.experimental.pallas import tpu as pltpu
from jax.experimental.pallas import tpu_sc as plsc

B = 16384
F = 26
K = 16
NW = 32
RPW = B // NW
CHUNK = 64
NCHUNK = RPW // CHUNK
IPC = CHUNK * F
IPW = RPW * F
GPC = IPC // 128
NVEC = IPC // K


V = 1000012
VPAD = -(-V // K) * K
NROW16 = VPAD // K


def _fm_body(x_hbm, emb_hbm, fc16_hbm, wb_hbm, out_hbm,
             idx_v, idx2_v, lane_v, rows_v, fcr_v, inter_v, out_v, wb_v, sem):
    wid = lax.axis_index("s") * 2 + lax.axis_index("c")

    pltpu.sync_copy(wb_hbm, wb_v)
    w_vec = wb_v[0, :]
    b_vec = wb_v[1, :]

    iota = lax.iota(jnp.int32, K)

    for chunk in range(NCHUNK):
        pltpu.sync_copy(
            x_hbm.at[pl.ds(wid * IPW + chunk * IPC, IPC)], idx_v)

        def shift_body(i, carry):
            v = idx_v[pl.ds(i * K, K)]
            idx2_v[pl.ds(i * K, K)] = lax.shift_right_logical(v, 4)
            lane_v[pl.ds(i * K, K)] = lax.bitwise_and(v, K - 1)
            return carry

        lax.fori_loop(0, NVEC, shift_body, 0, unroll=4)

        copies = []
        for j in range(GPC):
            copies.append(pltpu.async_copy(
                emb_hbm.at[idx_v.at[pl.ds(j * 128, 128)]],
                rows_v.at[pl.ds(j * 128, 128)], sem))
            copies.append(pltpu.async_copy(
                fc16_hbm.at[idx2_v.at[pl.ds(j * 128, 128)]],
                fcr_v.at[pl.ds(j * 128, 128)], sem))
        for c in copies:
            c.wait()

        def row_body(r, carry):
            base = r * F
            e = rows_v[base, :]
            acc = e
            accq = e * e
            for f in range(1, F):
                e = rows_v[base + f, :]
                acc = acc + e
                accq = accq + e * e
            inter_v[pl.ds(r * K, K)] = acc * acc - accq
            return carry

        lax.fori_loop(0, CHUNK, row_body, 0, unroll=2)

        for g in range(CHUNK // K):
            racc = w_vec * 0.0
            gb = g * K * K
            for k in range(K):
                racc = racc + plsc.load_gather(inter_v, [gb + k + iota * K])
            facc = w_vec * 0.0
            fb = g * K * F
            i26 = iota * F
            for f in range(F):
                t = fb + f + i26
                lane = plsc.load_gather(lane_v, [t])
                facc = facc + plsc.load_gather(fcr_v, [t, lane])
            z = facc * w_vec + b_vec + 0.5 * racc
            sig = 1.0 / (1.0 + jnp.exp(-z))
            out_v[pl.ds(chunk * CHUNK + g * K, K)] = sig

    pltpu.sync_copy(out_v, out_hbm.at[pl.ds(wid * RPW, RPW)])


@jax.jit
def _fm(x_flat, emb, fc16, wb):
    run = pl.kernel(
        _fm_body,
        out_type=jax.ShapeDtypeStruct((B,), jnp.float32),
        mesh=plsc.VectorSubcoreMesh(core_axis_name="c", subcore_axis_name="s"),
        compiler_params=pltpu.CompilerParams(
            needs_layout_passes=False, use_tc_tiling_on_sc=False),
        scratch_types=[
            pltpu.VMEM((IPC,), jnp.int32),
            pltpu.VMEM((IPC,), jnp.int32),
            pltpu.VMEM((IPC,), jnp.int32),
            pltpu.VMEM((IPC, K), jnp.float32),
            pltpu.VMEM((IPC, K), jnp.float32),
            pltpu.VMEM((CHUNK * K,), jnp.float32),
            pltpu.VMEM((RPW,), jnp.float32),
            pltpu.VMEM((2, K), jnp.float32),
            pltpu.SemaphoreType.DMA,
        ],
    )
    return run(x_flat, emb, fc16, wb)


def kernel(x, emb, fc, W, b):
    x_flat = x.reshape(-1).astype(jnp.int32)
    fc16 = jnp.pad(fc.reshape(-1), (0, VPAD - V)).reshape(NROW16, K)
    wb = jnp.concatenate(
        [jnp.full((1, K), W[0, 0], jnp.float32),
         jnp.full((1, K), b[0], jnp.float32)], axis=0)
    out = _fm(x_flat, emb, fc16, wb)
    return out.reshape(B, 1)


# speedup vs baseline: 1.0566x; 1.0566x over previous
import jax
import jax.numpy as jnp
from jax import lax
from jax.experimental import pallas as pl
from jax.experimental.pallas import tpu as pltpu
from jax.experimental.pallas import tpu_sc as plsc

B = 16384
F = 26
K = 16
NW = 32
RPW = B // NW
CHUNK = 64
NCHUNK = RPW // CHUNK
IPC = CHUNK * F
IPW = RPW * F
GPC = IPC // 128
NVEC = IPC // K


V = 1000012
VPAD = -(-V // K) * K
NROW16 = VPAD // K


def _fm_body(x_hbm, emb_hbm, fc16_hbm, wb_hbm, out_hbm,
             idx_v, idx2_v, lane_v, rows_v, fcr_v, inter_v, out_v, wb_v,
             sem0, sem1):
    wid = lax.axis_index("s") * 2 + lax.axis_index("c")
    sems = (sem0, sem1)

    pltpu.sync_copy(wb_hbm, wb_v)
    w_vec = wb_v[0, :]
    b_vec = wb_v[1, :]

    iota = lax.iota(jnp.int32, K)

    def stage(chunk, p):
        pltpu.sync_copy(
            x_hbm.at[pl.ds(wid * IPW + chunk * IPC, IPC)], idx_v.at[p])

        def shift_body(i, carry):
            v = idx_v[p, pl.ds(i * K, K)]
            idx2_v[p, pl.ds(i * K, K)] = lax.shift_right_logical(v, 4)
            lane_v[p, pl.ds(i * K, K)] = lax.bitwise_and(v, K - 1)
            return carry

        lax.fori_loop(0, NVEC, shift_body, 0, unroll=4)

        copies = []
        for j in range(GPC):
            copies.append(pltpu.async_copy(
                emb_hbm.at[idx_v.at[p].at[pl.ds(j * 128, 128)]],
                rows_v.at[p].at[pl.ds(j * 128, 128)], sems[p]))
            copies.append(pltpu.async_copy(
                fc16_hbm.at[idx2_v.at[p].at[pl.ds(j * 128, 128)]],
                fcr_v.at[p].at[pl.ds(j * 128, 128)], sems[p]))
        return copies

    copies = stage(0, 0)
    for chunk in range(NCHUNK):
        p = chunk % 2
        next_copies = stage(chunk + 1, 1 - p) if chunk + 1 < NCHUNK else []
        for c in copies:
            c.wait()
        copies = next_copies

        def row_body(r, carry):
            base = r * F
            e = rows_v[p, base, :]
            acc = e
            accq = e * e
            for f in range(1, F):
                e = rows_v[p, base + f, :]
                acc = acc + e
                accq = accq + e * e
            inter_v[pl.ds(r * K, K)] = acc * acc - accq
            return carry

        lax.fori_loop(0, CHUNK, row_body, 0, unroll=2)

        for g in range(CHUNK // K):
            racc = w_vec * 0.0
            gb = g * K * K
            for k in range(K):
                racc = racc + plsc.load_gather(inter_v, [gb + k + iota * K])
            facc = w_vec * 0.0
            fb = g * K * F
            i26 = iota * F
            for f in range(F):
                t = fb + f + i26
                lane = plsc.load_gather(lane_v.at[p], [t])
                facc = facc + plsc.load_gather(fcr_v.at[p], [t, lane])
            z = facc * w_vec + b_vec + 0.5 * racc
            sig = 1.0 / (1.0 + jnp.exp(-z))
            out_v[pl.ds(chunk * CHUNK + g * K, K)] = sig

    pltpu.sync_copy(out_v, out_hbm.at[pl.ds(wid * RPW, RPW)])


@jax.jit
def _fm(x_flat, emb, fc16, wb):
    run = pl.kernel(
        _fm_body,
        out_type=jax.ShapeDtypeStruct((B,), jnp.float32),
        mesh=plsc.VectorSubcoreMesh(core_axis_name="c", subcore_axis_name="s"),
        compiler_params=pltpu.CompilerParams(
            needs_layout_passes=False, use_tc_tiling_on_sc=False),
        scratch_types=[
            pltpu.VMEM((2, IPC), jnp.int32),
            pltpu.VMEM((2, IPC), jnp.int32),
            pltpu.VMEM((2, IPC), jnp.int32),
            pltpu.VMEM((2, IPC, K), jnp.float32),
            pltpu.VMEM((2, IPC, K), jnp.float32),
            pltpu.VMEM((CHUNK * K,), jnp.float32),
            pltpu.VMEM((RPW,), jnp.float32),
            pltpu.VMEM((2, K), jnp.float32),
            pltpu.SemaphoreType.DMA,
            pltpu.SemaphoreType.DMA,
        ],
    )
    return run(x_flat, emb, fc16, wb)


def kernel(x, emb, fc, W, b):
    x_flat = x.reshape(-1).astype(jnp.int32)
    fc16 = jnp.pad(fc.reshape(-1), (0, VPAD - V)).reshape(NROW16, K)
    wb = jnp.concatenate(
        [jnp.full((1, K), W[0, 0], jnp.float32),
         jnp.full((1, K), b[0], jnp.float32)], axis=0)
    out = _fm(x_flat, emb, fc16, wb)
    return out.reshape(B, 1)


# speedup vs baseline: 1.0610x; 1.0041x over previous
import jax
import jax.numpy as jnp
from jax import lax
from jax.experimental import pallas as pl
from jax.experimental.pallas import tpu as pltpu
from jax.experimental.pallas import tpu_sc as plsc

B = 16384
F = 26
K = 16
NW = 32
RPW = B // NW
CHUNK = 64
NCHUNK = RPW // CHUNK
IPC = CHUNK * F
IPW = RPW * F
GPC = IPC // 128
NVEC = IPC // K


V = 1000012
VPAD = -(-V // K) * K
NROW16 = VPAD // K


def _fm_body(x_hbm, emb_hbm, fc16_hbm, wb_hbm, out_hbm,
             idx_v, idx2_v, lane_v, rows_v, fcr_v, inter_v, out_v,
             wb_v, sem0, sem1):
    wid = lax.axis_index("s") * 2 + lax.axis_index("c")
    sems = (sem0, sem1)

    pltpu.sync_copy(wb_hbm, wb_v)
    b_vec = wb_v[0, :]
    zero_vec = b_vec * 0.0

    iota = lax.iota(jnp.int32, K)

    def stage(chunk, p):
        pltpu.sync_copy(
            x_hbm.at[pl.ds(wid * IPW + chunk * IPC, IPC)], idx_v.at[p])

        def shift_body(i, carry):
            v = idx_v[p, pl.ds(i * K, K)]
            idx2_v[p, pl.ds(i * K, K)] = lax.shift_right_logical(v, 4)
            lane_v[p, pl.ds(i * K, K)] = lax.bitwise_and(v, K - 1)
            return carry

        lax.fori_loop(0, NVEC, shift_body, 0, unroll=4)

        copies = []
        for j in range(GPC):
            copies.append(pltpu.async_copy(
                emb_hbm.at[idx_v.at[p].at[pl.ds(j * 128, 128)]],
                rows_v.at[p].at[pl.ds(j * 128, 128)], sems[p]))
            copies.append(pltpu.async_copy(
                fc16_hbm.at[idx2_v.at[p].at[pl.ds(j * 128, 128)]],
                fcr_v.at[p].at[pl.ds(j * 128, 128)], sems[p]))
        return copies

    copies = stage(0, 0)
    for chunk in range(NCHUNK):
        p = chunk % 2
        next_copies = stage(chunk + 1, 1 - p) if chunk + 1 < NCHUNK else []
        for c in copies:
            c.wait()
        copies = next_copies

        def row_body(r, carry):
            base = r * F
            e = rows_v[p, base, :]
            acc = e
            accq = e * e
            for f in range(1, F):
                e = rows_v[p, base + f, :]
                acc = acc + e
                accq = accq + e * e
            inter_v[pl.ds(r * K, K)] = acc * acc - accq
            return carry

        lax.fori_loop(0, CHUNK, row_body, 0, unroll=2)

        for g in range(CHUNK // K):
            racc = zero_vec
            gb = g * K * K
            for k in range(K):
                racc = racc + plsc.load_gather(inter_v, [gb + k + iota * K])
            facc = zero_vec
            fb = g * K * F
            i26 = iota * F
            for f in range(F):
                t = fb + f + i26
                lane = plsc.load_gather(lane_v.at[p], [t])
                facc = facc + plsc.load_gather(fcr_v.at[p], [t, lane])
            z = facc + b_vec + 0.5 * racc
            sig = 1.0 / (1.0 + jnp.exp(-z))
            out_v[pl.ds(chunk * CHUNK + g * K, K)] = sig

    pltpu.sync_copy(out_v, out_hbm.at[pl.ds(wid * RPW, RPW)])


@jax.jit
def _fm(x_flat, emb, fc, wb):
    run = pl.kernel(
        _fm_body,
        out_type=jax.ShapeDtypeStruct((B,), jnp.float32),
        mesh=plsc.VectorSubcoreMesh(core_axis_name="c", subcore_axis_name="s"),
        compiler_params=pltpu.CompilerParams(
            needs_layout_passes=False, use_tc_tiling_on_sc=False),
        scratch_types=[
            pltpu.VMEM((2, IPC), jnp.int32),
            pltpu.VMEM((2, IPC), jnp.int32),
            pltpu.VMEM((2, IPC), jnp.int32),
            pltpu.VMEM((2, IPC, K), jnp.float32),
            pltpu.VMEM((2, IPC, K), jnp.float32),
            pltpu.VMEM((CHUNK * K,), jnp.float32),
            pltpu.VMEM((RPW,), jnp.float32),
            pltpu.VMEM((1, K), jnp.float32),
            pltpu.SemaphoreType.DMA,
            pltpu.SemaphoreType.DMA,
        ],
    )
    return run(x_flat, emb, fc, wb)


def kernel(x, emb, fc, W, b):
    x_flat = x.reshape(-1).astype(jnp.int32)
    fc16 = jnp.pad(fc.reshape(-1) * W[0, 0], (0, VPAD - V)).reshape(NROW16, K)
    bb = jnp.full((1, K), b[0], jnp.float32)
    return _fm(x_flat, emb, fc16, bb).reshape(B, 1)
